# baseline (speedup 1.0000x reference)
import functools

import jax
import jax.numpy as jnp
from jax import lax
from jax.experimental import pallas as pl
from jax.experimental.pallas import tpu as pltpu
from jax.experimental.pallas import tpu_sc as plsc

B = 64
CAP = 2048
SEQ = 1024
D = 256
NC, NS = 2, 16
NW = NC * NS
BPW = B // NW
CHUNK = 128
LANES = 16
DEPTH = 3

_mesh = plsc.VectorSubcoreMesh(
    core_axis_name="c", subcore_axis_name="s", num_cores=NC, num_subcores=NS
)


@functools.partial(
    pl.kernel,
    out_type=jax.ShapeDtypeStruct((B * CAP, D), jnp.float32),
    mesh=_mesh,
    scratch_types=(
        [pltpu.VMEM((B + LANES,), jnp.int32)]
        + [pltpu.VMEM((CHUNK,), jnp.int32) for _ in range(DEPTH)]
        + [pltpu.VMEM((CHUNK, D), jnp.float32) for _ in range(DEPTH)]
        + [pltpu.SemaphoreType.DMA for _ in range(2 * DEPTH)]
    ),
    compiler_params=pltpu.CompilerParams(use_tc_tiling_on_sc=False),
)
def _scatter(buf_hbm, obs_hbm, idx_hbm, out_hbm, idx_v, *scratch):
    idx_refs = scratch[:DEPTH]
    data_refs = scratch[DEPTH:2 * DEPTH]
    sems_in = scratch[2 * DEPTH:3 * DEPTH]
    sems_out = scratch[3 * DEPTH:4 * DEPTH]

    wid = lax.axis_index("s") * NC + lax.axis_index("c")
    pltpu.sync_copy(idx_hbm, idx_v.at[pl.ds(0, B)])
    lanes = lax.iota(jnp.int32, LANES)

    starts = []
    for k in range(BPW):
        b = wid * BPW + k
        starts.append(idx_v[pl.ds(b, LANES)][0])

    jobs = []
    for k in range(BPW):
        b = wid * BPW + k
        i = starts[k]
        for c in range(SEQ // CHUNK):
            jobs.append(("obs", b, i, c))
        for c in range(SEQ // CHUNK):
            jobs.append(("buf", b, i, c))

    meta = []
    for (kind, b, i, c) in jobs:
        circ0 = i + c * CHUNK + (SEQ if kind == "buf" else 0)
        start = circ0 & (CAP - 1)
        wraps = start + CHUNK > CAP
        meta.append((kind, b, i, c, start, wraps))

    def fill_ids(s, b, start):
        for v in range(CHUNK // LANES):
            off = start + v * LANES + lanes
            idx_refs[s][pl.ds(v * LANES, LANES)] = b * CAP + (off & (CAP - 1))

    def start_out(j):
        kind, b, i, c, start, wraps = meta[j]
        s = j % DEPTH

        @pl.when(wraps)
        def _():
            pltpu.async_copy(data_refs[s], out_hbm.at[idx_refs[s]],
                             sems_out[s])

        @pl.when(jnp.logical_not(wraps))
        def _():
            pltpu.async_copy(data_refs[s],
                             out_hbm.at[pl.ds(b * CAP + start, CHUNK)],
                             sems_out[s])

    for j, (kind, b, i, c, start, wraps) in enumerate(meta):
        s = j % DEPTH
        if j >= DEPTH:
            pltpu.make_async_copy(
                data_refs[s], out_hbm.at[pl.ds(0, CHUNK)], sems_out[s]).wait()

        @pl.when(wraps)
        def _():
            fill_ids(s, b, start)

        if kind == "obs":
            pltpu.async_copy(obs_hbm.at[pl.ds(b * SEQ + c * CHUNK, CHUNK)],
                             data_refs[s], sems_in[s])
        else:
            @pl.when(wraps)
            def _():
                pltpu.async_copy(buf_hbm.at[idx_refs[s]], data_refs[s],
                                 sems_in[s])

            @pl.when(jnp.logical_not(wraps))
            def _():
                pltpu.async_copy(buf_hbm.at[pl.ds(b * CAP + start, CHUNK)],
                                 data_refs[s], sems_in[s])

        if j >= 1:
            sp = (j - 1) % DEPTH
            pltpu.make_async_copy(
                obs_hbm.at[pl.ds(0, CHUNK)], data_refs[sp],
                sems_in[sp]).wait()
            start_out(j - 1)

    j_last = len(meta) - 1
    sl = j_last % DEPTH
    pltpu.make_async_copy(
        obs_hbm.at[pl.ds(0, CHUNK)], data_refs[sl], sems_in[sl]).wait()
    start_out(j_last)
    for j in range(max(0, len(meta) - DEPTH), len(meta)):
        s = j % DEPTH
        pltpu.make_async_copy(
            data_refs[s], out_hbm.at[pl.ds(0, CHUNK)], sems_out[s]).wait()


def kernel(buffer, observation_sequence, index, size):
    del size
    buf2d = buffer.reshape(B * CAP, D)
    obs2d = observation_sequence.reshape(B * SEQ, D)
    out2d = _scatter(buf2d, obs2d, index)
    return out2d.reshape(B, CAP, D)


# speedup vs baseline: 3.0277x; 3.0277x over previous
import functools

import jax
import jax.numpy as jnp
from jax import lax
from jax.experimental import pallas as pl
from jax.experimental.pallas import tpu as pltpu
from jax.experimental.pallas import tpu_sc as plsc

B = 64
CAP = 2048
SEQ = 1024
D = 256
NC, NS = 2, 16
NW = NC * NS
BPW = B // NW
CHUNK = 128
LANES = 16
DEPTH = 3

_mesh = plsc.VectorSubcoreMesh(
    core_axis_name="c", subcore_axis_name="s", num_cores=NC, num_subcores=NS
)


@functools.partial(
    pl.kernel,
    out_type=jax.ShapeDtypeStruct((B * CAP, D), jnp.float32),
    mesh=_mesh,
    scratch_types=(
        [pltpu.VMEM((B + LANES,), jnp.int32)]
        + [pltpu.VMEM((CHUNK,), jnp.int32) for _ in range(DEPTH)]
        + [pltpu.VMEM((CHUNK, D), jnp.float32) for _ in range(DEPTH)]
        + [pltpu.SemaphoreType.DMA for _ in range(2 * DEPTH)]
    ),
)
def _scatter(buf_hbm, obs_hbm, idx_hbm, out_hbm, idx_v, *scratch):
    idx_refs = scratch[:DEPTH]
    data_refs = scratch[DEPTH:2 * DEPTH]
    sems_in = scratch[2 * DEPTH:3 * DEPTH]
    sems_out = scratch[3 * DEPTH:4 * DEPTH]

    wid = lax.axis_index("s") * NC + lax.axis_index("c")
    pltpu.sync_copy(idx_hbm, idx_v.at[pl.ds(0, B)])
    lanes = lax.iota(jnp.int32, LANES)

    def wait_in(s):
        pltpu.make_async_copy(
            obs_hbm.at[pl.ds(0, CHUNK)], data_refs[s], sems_in[s]).wait()

    def wait_out(s):
        pltpu.make_async_copy(
            data_refs[s], out_hbm.at[pl.ds(0, CHUNK)], sems_out[s]).wait()

    def fill_ids(s, base, circ0, mask):
        for v in range(CHUNK // LANES):
            off = circ0 + v * LANES + lanes
            idx_refs[s][pl.ds(v * LANES, LANES)] = base + (off & mask)

    jobs = []
    for k in range(BPW):
        b = wid * BPW + k
        i = idx_v[pl.ds(b, LANES)][0]

        for c in range(CAP // CHUNK):
            s_c = (c * CHUNK - i) & (CAP - 1)
            is_obs = s_c <= SEQ - CHUNK

            def start_in(s, b=b, c=c, s_c=s_c, is_obs=is_obs):
                @pl.when(is_obs)
                def _():
                    fill_ids(s, b * SEQ, s_c, SEQ - 1)
                    pltpu.async_copy(obs_hbm.at[idx_refs[s]],
                                     data_refs[s], sems_in[s])

                @pl.when(jnp.logical_not(is_obs))
                def _():
                    pltpu.async_copy(
                        buf_hbm.at[pl.ds(
                            pl.multiple_of(b * CAP + c * CHUNK, CHUNK),
                            CHUNK)],
                        data_refs[s], sems_in[s])

            def start_out(s, b=b, c=c):
                pltpu.async_copy(
                    data_refs[s],
                    out_hbm.at[pl.ds(
                        pl.multiple_of(b * CAP + c * CHUNK, CHUNK), CHUNK)],
                    sems_out[s])

            jobs.append((start_in, start_out))

        jobs.append("drain")

        for which in range(2):
            src0 = 0 if which == 0 else SEQ - CHUNK
            circ0 = i + src0

            def start_in(s, b=b, src0=src0):
                pltpu.async_copy(
                    obs_hbm.at[pl.ds(
                        pl.multiple_of(b * SEQ + src0, CHUNK), CHUNK)],
                    data_refs[s], sems_in[s])

            def start_out(s, b=b, circ0=circ0):
                fill_ids(s, b * CAP, circ0, CAP - 1)
                pltpu.async_copy(data_refs[s], out_hbm.at[idx_refs[s]],
                                 sems_out[s])

            jobs.append((start_in, start_out))
        jobs.append("drain")

    slot_of = {}
    out_pending = []
    in_pending = []
    slot_busy = {}
    jreal = 0

    def flush_in():
        while in_pending:
            jj = in_pending.pop(0)
            wait_in(slot_of[jj])
            jobs[jj][1](slot_of[jj])
            out_pending.append(jj)

    for j, job in enumerate(jobs):
        if job == "drain":
            flush_in()
            while out_pending:
                wait_out(slot_of[out_pending.pop(0)])
            slot_busy.clear()
            continue
        s = jreal % DEPTH
        jreal += 1
        if s in slot_busy:
            jj = slot_busy[s]
            if jj in out_pending:
                while out_pending:
                    j0 = out_pending.pop(0)
                    wait_out(slot_of[j0])
                    if j0 == jj:
                        break
        slot_of[j] = s
        job[0](s)
        in_pending.append(j)
        slot_busy[s] = j
        while len(in_pending) > 1:
            jj = in_pending.pop(0)
            wait_in(slot_of[jj])
            jobs[jj][1](slot_of[jj])
            out_pending.append(jj)

    flush_in()
    while out_pending:
        wait_out(slot_of[out_pending.pop(0)])


def kernel(buffer, observation_sequence, index, size):
    del size
    buf2d = buffer.reshape(B * CAP, D)
    obs2d = observation_sequence.reshape(B * SEQ, D)
    out2d = _scatter(buf2d, obs2d, index)
    return out2d.reshape(B, CAP, D)


# speedup vs baseline: 3.0719x; 1.0146x over previous
import functools

import jax
import jax.numpy as jnp
from jax import lax
from jax.experimental import pallas as pl
from jax.experimental.pallas import tpu as pltpu
from jax.experimental.pallas import tpu_sc as plsc

B = 64
CAP = 2048
SEQ = 1024
D = 256
NC, NS = 2, 16
NW = NC * NS
BPW = B // NW
CHUNK = 128
LANES = 16
DEPTH = 3

_mesh = plsc.VectorSubcoreMesh(
    core_axis_name="c", subcore_axis_name="s", num_cores=NC, num_subcores=NS
)


@functools.partial(
    pl.kernel,
    out_type=jax.ShapeDtypeStruct((B * CAP, D), jnp.float32),
    mesh=_mesh,
    scratch_types=(
        [pltpu.VMEM((B + LANES,), jnp.int32)]
        + [pltpu.VMEM((CHUNK,), jnp.int32) for _ in range(DEPTH)]
        + [pltpu.VMEM((CHUNK, D), jnp.float32) for _ in range(DEPTH)]
        + [pltpu.SemaphoreType.DMA for _ in range(2 * DEPTH)]
    ),
)
def _scatter(buf_hbm, obs_hbm, idx_hbm, out_hbm, idx_v, *scratch):
    idx_refs = scratch[:DEPTH]
    data_refs = scratch[DEPTH:2 * DEPTH]
    sems_in = scratch[2 * DEPTH:3 * DEPTH]
    sems_out = scratch[3 * DEPTH:4 * DEPTH]

    wid = lax.axis_index("s") * NC + lax.axis_index("c")
    pltpu.sync_copy(idx_hbm, idx_v.at[pl.ds(0, B)])
    lanes = lax.iota(jnp.int32, LANES)

    def wait_in(s):
        pltpu.make_async_copy(
            obs_hbm.at[pl.ds(0, CHUNK)], data_refs[s], sems_in[s]).wait()

    def wait_out(s):
        pltpu.make_async_copy(
            data_refs[s], out_hbm.at[pl.ds(0, CHUNK)], sems_out[s]).wait()

    def fill_ids(s, base, circ0):
        for v in range(CHUNK // LANES):
            off = circ0 + v * LANES + lanes
            idx_refs[s][pl.ds(v * LANES, LANES)] = base + (off & (CAP - 1))

    jobs = []
    for k in range(BPW):
        b = wid * BPW + k
        i = idx_v[pl.ds(b, LANES)][0]

        for c in range(SEQ // CHUNK):
            def start_in(s, b=b, c=c):
                pltpu.async_copy(
                    obs_hbm.at[pl.ds(
                        pl.multiple_of(b * SEQ + c * CHUNK, CHUNK), CHUNK)],
                    data_refs[s], sems_in[s])

            def start_out(s, b=b, i=i, c=c):
                fill_ids(s, b * CAP, i + c * CHUNK)
                pltpu.async_copy(data_refs[s], out_hbm.at[idx_refs[s]],
                                 sems_out[s])

            jobs.append((None, start_in, start_out))

        for c in range(CAP // CHUNK):
            s_c = (c * CHUNK - i) & (CAP - 1)
            is_buf = s_c >= SEQ

            def start_in(s, b=b, c=c):
                pltpu.async_copy(
                    buf_hbm.at[pl.ds(
                        pl.multiple_of(b * CAP + c * CHUNK, CHUNK), CHUNK)],
                    data_refs[s], sems_in[s])

            def start_out(s, b=b, c=c):
                pltpu.async_copy(
                    data_refs[s],
                    out_hbm.at[pl.ds(
                        pl.multiple_of(b * CAP + c * CHUNK, CHUNK), CHUNK)],
                    sems_out[s])

            jobs.append((is_buf, start_in, start_out))

        for which in range(2):
            circ0 = i - CHUNK if which == 0 else i + SEQ

            def start_in(s, b=b, circ0=circ0):
                fill_ids(s, b * CAP, circ0)
                pltpu.async_copy(buf_hbm.at[idx_refs[s]], data_refs[s],
                                 sems_in[s])

            def start_out(s):
                pltpu.async_copy(data_refs[s], out_hbm.at[idx_refs[s]],
                                 sems_out[s])

            jobs.append((None, start_in, start_out))

    def guarded(cond, fn, *args):
        if cond is None:
            fn(*args)
        else:
            pl.when(cond)(lambda: fn(*args))

    slots = {}
    out_pending = []
    in_pending = []
    for j, (cond, start_in, start_out) in enumerate(jobs):
        s = j % DEPTH
        if j >= DEPTH:
            while out_pending and out_pending[0] <= j - DEPTH:
                jj = out_pending.pop(0)
                guarded(jobs[jj][0], wait_out, slots[jj])
        slots[j] = s
        guarded(cond, start_in, s)
        in_pending.append(j)
        while len(in_pending) > 1:
            jj = in_pending.pop(0)
            guarded(jobs[jj][0], wait_in, slots[jj])
            guarded(jobs[jj][0], jobs[jj][2], slots[jj])
            out_pending.append(jj)

    while in_pending:
        jj = in_pending.pop(0)
        guarded(jobs[jj][0], wait_in, slots[jj])
        guarded(jobs[jj][0], jobs[jj][2], slots[jj])
        out_pending.append(jj)
    while out_pending:
        jj = out_pending.pop(0)
        guarded(jobs[jj][0], wait_out, slots[jj])


def kernel(buffer, observation_sequence, index, size):
    del size
    buf2d = buffer.reshape(B * CAP, D)
    obs2d = observation_sequence.reshape(B * SEQ, D)
    out2d = _scatter(buf2d, obs2d, index)
    return out2d.reshape(B, CAP, D)


# speedup vs baseline: 3.2235x; 1.0493x over previous
import functools

import jax
import jax.numpy as jnp
from jax import lax
from jax.experimental import pallas as pl
from jax.experimental.pallas import tpu as pltpu
from jax.experimental.pallas import tpu_sc as plsc

B = 64
CAP = 2048
SEQ = 1024
D = 256
NC, NS = 2, 16
NW = NC * NS
BPW = B // NW
CHUNK = 128
LANES = 16
DEPTH = 3

_mesh = plsc.VectorSubcoreMesh(
    core_axis_name="c", subcore_axis_name="s", num_cores=NC, num_subcores=NS
)


@functools.partial(
    pl.kernel,
    out_type=jax.ShapeDtypeStruct((B * CAP, D), jnp.float32),
    mesh=_mesh,
    scratch_types=(
        [pltpu.VMEM((B + LANES,), jnp.int32)]
        + [pltpu.VMEM((CHUNK,), jnp.int32) for _ in range(DEPTH)]
        + [pltpu.VMEM((CHUNK, D), jnp.float32) for _ in range(DEPTH)]
        + [pltpu.SemaphoreType.DMA for _ in range(2 * DEPTH)]
    ),
)
def _scatter(buf_hbm, obs_hbm, idx_hbm, out_hbm, idx_v, *scratch):
    idx_refs = scratch[:DEPTH]
    data_refs = scratch[DEPTH:2 * DEPTH]
    sems_in = scratch[2 * DEPTH:3 * DEPTH]
    sems_out = scratch[3 * DEPTH:4 * DEPTH]

    wid = lax.axis_index("s") * NC + lax.axis_index("c")
    pltpu.sync_copy(idx_hbm, idx_v.at[pl.ds(0, B)])
    lanes = lax.iota(jnp.int32, LANES)

    def wait_in(s):
        pltpu.make_async_copy(
            obs_hbm.at[pl.ds(0, CHUNK)], data_refs[s], sems_in[s]).wait()

    def wait_out(s):
        pltpu.make_async_copy(
            data_refs[s], out_hbm.at[pl.ds(0, CHUNK)], sems_out[s]).wait()

    def fill_ids(s, base, circ0):
        for v in range(CHUNK // LANES):
            off = circ0 + v * LANES + lanes
            idx_refs[s][pl.ds(v * LANES, LANES)] = base + (off & (CAP - 1))

    jobs = []
    for k in range(BPW):
        b = wid * BPW + k
        i = idx_v[pl.ds(b, LANES)][0]

        for c in range(SEQ // CHUNK):
            def start_in(s, b=b, c=c):
                pltpu.async_copy(
                    obs_hbm.at[pl.ds(
                        pl.multiple_of(b * SEQ + c * CHUNK, CHUNK), CHUNK)],
                    data_refs[s], sems_in[s])

            def start_out(s, b=b, i=i, c=c):
                fill_ids(s, b * CAP, i + c * CHUNK)
                pltpu.async_copy(data_refs[s], out_hbm.at[idx_refs[s]],
                                 sems_out[s])

            jobs.append((None, start_in, start_out))

        for c in range(CAP // CHUNK):
            s_c = (c * CHUNK - i) & (CAP - 1)
            is_buf = jnp.logical_and(s_c >= SEQ, s_c <= CAP - CHUNK)

            def start_in(s, b=b, c=c):
                pltpu.async_copy(
                    buf_hbm.at[pl.ds(
                        pl.multiple_of(b * CAP + c * CHUNK, CHUNK), CHUNK)],
                    data_refs[s], sems_in[s])

            def start_out(s, b=b, c=c):
                pltpu.async_copy(
                    data_refs[s],
                    out_hbm.at[pl.ds(
                        pl.multiple_of(b * CAP + c * CHUNK, CHUNK), CHUNK)],
                    sems_out[s])

            jobs.append((is_buf, start_in, start_out))

        for which in range(2):
            circ0 = i - CHUNK if which == 0 else i + SEQ

            def start_in(s, b=b, circ0=circ0):
                fill_ids(s, b * CAP, circ0)
                pltpu.async_copy(buf_hbm.at[idx_refs[s]], data_refs[s],
                                 sems_in[s])

            def start_out(s):
                pltpu.async_copy(data_refs[s], out_hbm.at[idx_refs[s]],
                                 sems_out[s])

            jobs.append((None, start_in, start_out))

    def guarded(cond, fn, *args):
        if cond is None:
            fn(*args)
        else:
            pl.when(cond)(lambda: fn(*args))

    slots = {}
    out_pending = []
    in_pending = []
    for j, (cond, start_in, start_out) in enumerate(jobs):
        s = j % DEPTH
        if j >= DEPTH:
            while out_pending and out_pending[0] <= j - DEPTH:
                jj = out_pending.pop(0)
                guarded(jobs[jj][0], wait_out, slots[jj])
        slots[j] = s
        guarded(cond, start_in, s)
        in_pending.append(j)
        while len(in_pending) > 1:
            jj = in_pending.pop(0)
            guarded(jobs[jj][0], wait_in, slots[jj])
            guarded(jobs[jj][0], jobs[jj][2], slots[jj])
            out_pending.append(jj)

    while in_pending:
        jj = in_pending.pop(0)
        guarded(jobs[jj][0], wait_in, slots[jj])
        guarded(jobs[jj][0], jobs[jj][2], slots[jj])
        out_pending.append(jj)
    while out_pending:
        jj = out_pending.pop(0)
        guarded(jobs[jj][0], wait_out, slots[jj])


def kernel(buffer, observation_sequence, index, size):
    del size
    buf2d = buffer.reshape(B * CAP, D)
    obs2d = observation_sequence.reshape(B * SEQ, D)
    out2d = _scatter(buf2d, obs2d, index)
    return out2d.reshape(B, CAP, D)


# speedup vs baseline: 3.3253x; 1.0316x over previous
import functools

import jax
import jax.numpy as jnp
from jax import lax
from jax.experimental import pallas as pl
from jax.experimental.pallas import tpu as pltpu
from jax.experimental.pallas import tpu_sc as plsc

B = 64
CAP = 2048
SEQ = 1024
D = 256
NC, NS = 2, 16
NW = NC * NS
BPW = B // NW
CHUNK = 64
LANES = 16
DEPTH = 6

_mesh = plsc.VectorSubcoreMesh(
    core_axis_name="c", subcore_axis_name="s", num_cores=NC, num_subcores=NS
)


@functools.partial(
    pl.kernel,
    out_type=jax.ShapeDtypeStruct((B * CAP, D), jnp.float32),
    mesh=_mesh,
    scratch_types=(
        [pltpu.VMEM((B + LANES,), jnp.int32)]
        + [pltpu.VMEM((CHUNK,), jnp.int32) for _ in range(DEPTH)]
        + [pltpu.VMEM((CHUNK, D), jnp.float32) for _ in range(DEPTH)]
        + [pltpu.SemaphoreType.DMA for _ in range(2 * DEPTH)]
    ),
)
def _scatter(buf_hbm, obs_hbm, idx_hbm, out_hbm, idx_v, *scratch):
    idx_refs = scratch[:DEPTH]
    data_refs = scratch[DEPTH:2 * DEPTH]
    sems_in = scratch[2 * DEPTH:3 * DEPTH]
    sems_out = scratch[3 * DEPTH:4 * DEPTH]

    wid = lax.axis_index("s") * NC + lax.axis_index("c")
    pltpu.sync_copy(idx_hbm, idx_v.at[pl.ds(0, B)])
    lanes = lax.iota(jnp.int32, LANES)

    def wait_in(s):
        pltpu.make_async_copy(
            obs_hbm.at[pl.ds(0, CHUNK)], data_refs[s], sems_in[s]).wait()

    def wait_out(s):
        pltpu.make_async_copy(
            data_refs[s], out_hbm.at[pl.ds(0, CHUNK)], sems_out[s]).wait()

    def fill_ids(s, base, circ0):
        for v in range(CHUNK // LANES):
            off = circ0 + v * LANES + lanes
            idx_refs[s][pl.ds(v * LANES, LANES)] = base + (off & (CAP - 1))

    jobs = []
    for k in range(BPW):
        b = wid * BPW + k
        i = idx_v[pl.ds(b, LANES)][0]

        for c in range(SEQ // CHUNK):
            def start_in(s, b=b, c=c):
                pltpu.async_copy(
                    obs_hbm.at[pl.ds(
                        pl.multiple_of(b * SEQ + c * CHUNK, CHUNK), CHUNK)],
                    data_refs[s], sems_in[s])

            def start_out(s, b=b, i=i, c=c):
                fill_ids(s, b * CAP, i + c * CHUNK)
                pltpu.async_copy(data_refs[s], out_hbm.at[idx_refs[s]],
                                 sems_out[s])

            jobs.append((None, start_in, start_out))

        for c in range(CAP // CHUNK):
            s_c = (c * CHUNK - i) & (CAP - 1)
            is_buf = jnp.logical_and(s_c >= SEQ, s_c <= CAP - CHUNK)

            def start_in(s, b=b, c=c):
                pltpu.async_copy(
                    buf_hbm.at[pl.ds(
                        pl.multiple_of(b * CAP + c * CHUNK, CHUNK), CHUNK)],
                    data_refs[s], sems_in[s])

            def start_out(s, b=b, c=c):
                pltpu.async_copy(
                    data_refs[s],
                    out_hbm.at[pl.ds(
                        pl.multiple_of(b * CAP + c * CHUNK, CHUNK), CHUNK)],
                    sems_out[s])

            jobs.append((is_buf, start_in, start_out))

        for which in range(2):
            circ0 = i - CHUNK if which == 0 else i + SEQ

            def start_in(s, b=b, circ0=circ0):
                fill_ids(s, b * CAP, circ0)
                pltpu.async_copy(buf_hbm.at[idx_refs[s]], data_refs[s],
                                 sems_in[s])

            def start_out(s):
                pltpu.async_copy(data_refs[s], out_hbm.at[idx_refs[s]],
                                 sems_out[s])

            jobs.append((None, start_in, start_out))

    def guarded(cond, fn, *args):
        if cond is None:
            fn(*args)
        else:
            pl.when(cond)(lambda: fn(*args))

    slots = {}
    out_pending = []
    in_pending = []
    for j, (cond, start_in, start_out) in enumerate(jobs):
        s = j % DEPTH
        if j >= DEPTH:
            while out_pending and out_pending[0] <= j - DEPTH:
                jj = out_pending.pop(0)
                guarded(jobs[jj][0], wait_out, slots[jj])
        slots[j] = s
        guarded(cond, start_in, s)
        in_pending.append(j)
        while len(in_pending) > 2:
            jj = in_pending.pop(0)
            guarded(jobs[jj][0], wait_in, slots[jj])
            guarded(jobs[jj][0], jobs[jj][2], slots[jj])
            out_pending.append(jj)

    while in_pending:
        jj = in_pending.pop(0)
        guarded(jobs[jj][0], wait_in, slots[jj])
        guarded(jobs[jj][0], jobs[jj][2], slots[jj])
        out_pending.append(jj)
    while out_pending:
        jj = out_pending.pop(0)
        guarded(jobs[jj][0], wait_out, slots[jj])


def kernel(buffer, observation_sequence, index, size):
    del size
    buf2d = buffer.reshape(B * CAP, D)
    obs2d = observation_sequence.reshape(B * SEQ, D)
    out2d = _scatter(buf2d, obs2d, index)
    return out2d.reshape(B, CAP, D)
